# baseline (speedup 1.0000x reference)
import jax
import jax.numpy as jnp
from jax.experimental import pallas as pl
from jax.experimental.pallas import tpu as pltpu
from jax.experimental.pallas import tpu_sc as plsc

_INNER_PRUNE = 0.3
_CROSS_PRUNE = 0.1
_HI = jax.lax.Precision.HIGHEST

_B = 128
_N = 78
_T = 10
_GN = _T + _N
_D = 128
_E = 1248
_GB = 8
_NPROG = _B // _GB
_R = _GB * _GN

_NC = 2
_NS = 16
_GPW = _B // (_NC * _NS)
_GG = _GN * _GN


def _adj_sc(ei_hbm, a_hbm, ei_v, acc_v):
    wid = jax.lax.axis_index("s") * _NC + jax.lax.axis_index("c")
    base = wid * _GPW
    ones = jnp.ones((16,), jnp.float32)
    zeros = jnp.zeros((16,), jnp.float32)
    for gg in range(_GPW):
        g = base + gg
        pltpu.sync_copy(ei_hbm.at[g], ei_v)

        def zbody(j, c):
            acc_v[pl.ds(pl.multiple_of(j * 16, 16), 16)] = zeros
            return c
        jax.lax.fori_loop(0, _GG // 16, zbody, 0)

        def ebody(j, c):
            o = pl.multiple_of(j * 16, 16)
            src = ei_v[0, pl.ds(o, 16)]
            dst = ei_v[1, pl.ds(o, 16)]
            plsc.addupdate_scatter(acc_v, [dst * _GN + src], ones)
            return c
        jax.lax.fori_loop(0, _E // 16, ebody, 0)

        pltpu.sync_copy(acc_v, a_hbm.at[g])


def _kern(x_ref, a_ref, tok_ref, rcross_ref, ccols_ref,
          wq1, bq1, wk1, bk1, wv1, bv1, ws1, bs1,
          wq2, bq2, wk2, bk2, wv2, bv2, ws2, bs2,
          waT, ba, out_ref,
          qr, kr, vr, sr, hr, lwr, pr):
    tok = tok_ref[:]
    ones_den = jnp.ones((_GN, _D), jnp.float32)

    g_tt = jax.lax.dot_general(tok, tok, (((1,), (1,)), ((), ())),
                               precision=_HI)
    wtt = jnp.where(jax.nn.sigmoid(g_tt) >= _INNER_PRUNE, 1.0, 0.0)
    wttpad = jnp.pad(wtt, ((_N, 0), (_N, 0)))

    xf = x_ref[:].reshape(_R, _D)
    zc = jax.lax.dot(xf, rcross_ref[:], precision=_HI)
    cw = jnp.where(jax.nn.sigmoid(zc) >= _CROSS_PRUNE, 1.0, 0.0)
    cw = cw.reshape(_GB, _GN, _GN) * ccols_ref[:][None]
    wall = cw + wttpad[None]

    for g in range(_GB):
        lwr[g] = jnp.log(a_ref[g] + wall[g])

    qr[:] = jax.lax.dot(xf, wq1[:], precision=_HI) + bq1[:]
    kr[:] = jax.lax.dot(xf, wk1[:], precision=_HI) + bk1[:]
    vr[:] = jax.lax.dot(xf, wv1[:], precision=_HI) + bv1[:]
    sr[:] = jax.lax.dot(xf, ws1[:], precision=_HI) + bs1[:]

    for g in range(_GB):
        r0 = g * _GN
        qg = qr[r0:r0 + _GN]
        kg = kr[r0:r0 + _GN]
        L = jax.lax.dot_general(qg, kg, (((1,), (1,)), ((), ())),
                                precision=_HI)
        ex = jnp.exp(L + lwr[g])
        agg = jax.lax.dot(ex, vr[r0:r0 + _GN], precision=_HI)
        den = jax.lax.dot(ex, ones_den, precision=_HI)
        o = agg / jnp.maximum(den, 1e-16) + sr[r0:r0 + _GN]
        hr[r0:r0 + _GN, :] = jnp.where(o >= 0.0, o, 0.01 * o)

    hf = hr[:]
    qr[:] = jax.lax.dot(hf, wq2[:], precision=_HI) + bq2[:]
    kr[:] = jax.lax.dot(hf, wk2[:], precision=_HI) + bk2[:]
    vr[:] = jax.lax.dot(hf, wv2[:], precision=_HI) + bv2[:]
    sr[:] = jax.lax.dot(hf, ws2[:], precision=_HI) + bs2[:]

    for g in range(_GB):
        r0 = g * _GN
        qg = qr[r0:r0 + _GN]
        kg = kr[r0:r0 + _GN]
        L = jax.lax.dot_general(qg, kg, (((1,), (1,)), ((), ())),
                                precision=_HI)
        ex = jnp.exp(L + lwr[g])
        agg = jax.lax.dot(ex, vr[r0:r0 + _GN], precision=_HI)
        den = jax.lax.dot(ex, ones_den, precision=_HI)
        o = agg / jnp.maximum(den, 1e-16) + sr[r0:r0 + _GN]
        pr[g:g + 1, :] = jnp.sum(o, axis=0, keepdims=True)

    Z = jax.lax.dot(pr[:], waT[:], precision=_HI) + ba[:]
    mz = jnp.max(Z, axis=1, keepdims=True)
    ez = jnp.exp(Z - mz)
    out_ref[:] = ez / jnp.sum(ez, axis=1, keepdims=True)


def kernel(x, edge_index, tokens,
           Wq1, bq1, Wk1, bk1, Wv1, bv1, Ws1, bs1,
           Wq2, bq2, Wk2, bk2, Wv2, bv2, Ws2, bs2, Wa, ba):
    C = Wa.shape[0]
    inv = 1.0 / jnp.sqrt(jnp.float32(_D))
    xb = jnp.concatenate(
        [x, jnp.broadcast_to(tokens[None], (_B, _T, _D))], axis=1)
    ei = edge_index.astype(jnp.int32)

    mesh = plsc.VectorSubcoreMesh(core_axis_name="c", subcore_axis_name="s")
    adj = pl.kernel(
        _adj_sc,
        out_type=jax.ShapeDtypeStruct((_B, _GG), jnp.float32),
        mesh=mesh,
        scratch_types=[
            pltpu.VMEM((2, _E), jnp.int32),
            pltpu.VMEM((_GG,), jnp.float32),
        ],
        compiler_params=pltpu.CompilerParams(needs_layout_passes=False),
    )(ei)
    A3 = adj.reshape(_B, _GN, _GN)

    rcross = jnp.concatenate(
        [jnp.zeros((_D, _N), jnp.float32), tokens.T], axis=1)
    ccols = jnp.zeros((_GN, _GN), jnp.float32)
    ccols = ccols.at[:_N, _N:].set(1.0)

    args = [xb, A3, tokens, rcross, ccols]
    for w, b, sc in ((Wq1, bq1, inv), (Wk1, bk1, 1.0), (Wv1, bv1, 1.0),
                     (Ws1, bs1, 1.0), (Wq2, bq2, inv), (Wk2, bk2, 1.0),
                     (Wv2, bv2, 1.0), (Ws2, bs2, 1.0)):
        args.append(w.T * sc)
        args.append(b.reshape(1, -1) * sc)
    args.append(Wa.T / jnp.float32(_GN))
    args.append(ba.reshape(1, -1))

    full = lambda i: (0, 0)
    in_specs = [
        pl.BlockSpec((_GB, _GN, _D), lambda i: (i, 0, 0)),
        pl.BlockSpec((_GB, _GN, _GN), lambda i: (i, 0, 0)),
        pl.BlockSpec((_T, _D), full),
        pl.BlockSpec((_D, _GN), full),
        pl.BlockSpec((_GN, _GN), full),
    ]
    for _ in range(8):
        in_specs.append(pl.BlockSpec((_D, _D), full))
        in_specs.append(pl.BlockSpec((1, _D), full))
    in_specs.append(pl.BlockSpec((_D, C), full))
    in_specs.append(pl.BlockSpec((1, C), full))

    f32 = jnp.float32
    scratch = [
        pltpu.VMEM((_R, _D), f32),
        pltpu.VMEM((_R, _D), f32),
        pltpu.VMEM((_R, _D), f32),
        pltpu.VMEM((_R, _D), f32),
        pltpu.VMEM((_R, _D), f32),
        pltpu.VMEM((_GB, _GN, _GN), f32),
        pltpu.VMEM((_GB, _D), f32),
    ]
    return pl.pallas_call(
        _kern,
        grid=(_NPROG,),
        in_specs=in_specs,
        out_specs=pl.BlockSpec((_GB, C), lambda i: (i, 0)),
        out_shape=jax.ShapeDtypeStruct((_B, C), jnp.float32),
        scratch_shapes=scratch,
        compiler_params=pltpu.CompilerParams(
            dimension_semantics=("parallel",)),
    )(*args)


# speedup vs baseline: 1.2859x; 1.2859x over previous
import jax
import jax.numpy as jnp
from jax.experimental import pallas as pl
from jax.experimental.pallas import tpu as pltpu
from jax.experimental.pallas import tpu_sc as plsc

_INNER_PRUNE = 0.3
_CROSS_PRUNE = 0.1
_HI = jax.lax.Precision.HIGHEST

_B = 128
_N = 78
_T = 10
_GN = _T + _N
_D = 128
_E = 1248
_GB = 8
_NPROG = _B // _GB
_R = _GB * _GN

_NC = 2
_NS = 16
_GPW = _B // (_NC * _NS)
_GG = _GN * _GN


def _adj_sc(ei_hbm, a_hbm, ei_v, acc_v):
    wid = jax.lax.axis_index("s") * _NC + jax.lax.axis_index("c")
    base = wid * _GPW
    ones = jnp.ones((16,), jnp.float32)
    zeros = jnp.zeros((16,), jnp.float32)
    for gg in range(_GPW):
        g = base + gg
        pltpu.sync_copy(ei_hbm.at[g], ei_v)

        def zbody(j, c):
            acc_v[pl.ds(pl.multiple_of(j * 16, 16), 16)] = zeros
            return c
        jax.lax.fori_loop(0, _GG // 16, zbody, 0)

        def ebody(j, c):
            o = pl.multiple_of(j * 16, 16)
            src = ei_v[0, pl.ds(o, 16)]
            dst = ei_v[1, pl.ds(o, 16)]
            plsc.addupdate_scatter(acc_v, [dst * _GN + src], ones)
            return c
        jax.lax.fori_loop(0, _E // 16, ebody, 0)

        pltpu.sync_copy(acc_v, a_hbm.at[g])


def _kern(x_ref, a_ref, tok_ref, rcross_ref, ccols_ref,
          wA1, bA1, wA2, bA2,
          waT, ba, out_ref,
          pscr, var, hr, lwr, pr):
    tok = tok_ref[:]

    g_tt = jax.lax.dot_general(tok, tok, (((1,), (1,)), ((), ())),
                               precision=_HI)
    wtt = jnp.where(jax.nn.sigmoid(g_tt) >= _INNER_PRUNE, 1.0, 0.0)
    wttpad = jnp.pad(wtt, ((_N, 0), (_N, 0)))

    xf = x_ref[:].reshape(_R, _D)
    zc = jax.lax.dot(xf, rcross_ref[:], precision=_HI)
    cw = jnp.where(jax.nn.sigmoid(zc) >= _CROSS_PRUNE, 1.0, 0.0)
    cw = cw.reshape(_GB, _GN, _GN) * ccols_ref[:][None]
    wall = cw + wttpad[None]

    for g in range(_GB):
        lwr[g] = jnp.log(a_ref[g] + wall[g])

    var[:, _D:] = jnp.ones((_R, _D), jnp.float32)
    pscr[:] = jax.lax.dot(xf, wA1[:], precision=_HI) + bA1[:]
    var[:, :_D] = pscr[:, 2 * _D:3 * _D]

    for g in range(_GB):
        r0 = g * _GN
        qg = pscr[r0:r0 + _GN, 0:_D]
        kg = pscr[r0:r0 + _GN, _D:2 * _D]
        L = jax.lax.dot_general(qg, kg, (((1,), (1,)), ((), ())),
                                precision=_HI)
        ex = jnp.exp(L + lwr[g])
        ad = jax.lax.dot(ex, var[r0:r0 + _GN], precision=_HI)
        o = (ad[:, :_D] / jnp.maximum(ad[:, _D:], 1e-16)
             + pscr[r0:r0 + _GN, 3 * _D:])
        hr[r0:r0 + _GN, :] = jnp.where(o >= 0.0, o, 0.01 * o)

    hf = hr[:]
    pscr[:] = jax.lax.dot(hf, wA2[:], precision=_HI) + bA2[:]
    var[:, :_D] = pscr[:, 2 * _D:3 * _D]

    for g in range(_GB):
        r0 = g * _GN
        qg = pscr[r0:r0 + _GN, 0:_D]
        kg = pscr[r0:r0 + _GN, _D:2 * _D]
        L = jax.lax.dot_general(qg, kg, (((1,), (1,)), ((), ())),
                                precision=_HI)
        ex = jnp.exp(L + lwr[g])
        ad = jax.lax.dot(ex, var[r0:r0 + _GN], precision=_HI)
        o = (ad[:, :_D] / jnp.maximum(ad[:, _D:], 1e-16)
             + pscr[r0:r0 + _GN, 3 * _D:])
        pr[g:g + 1, :] = jnp.sum(o, axis=0, keepdims=True)

    Z = jax.lax.dot(pr[:], waT[:], precision=_HI) + ba[:]
    mz = jnp.max(Z, axis=1, keepdims=True)
    ez = jnp.exp(Z - mz)
    out_ref[:] = ez / jnp.sum(ez, axis=1, keepdims=True)


def kernel(x, edge_index, tokens,
           Wq1, bq1, Wk1, bk1, Wv1, bv1, Ws1, bs1,
           Wq2, bq2, Wk2, bk2, Wv2, bv2, Ws2, bs2, Wa, ba):
    C = Wa.shape[0]
    inv = 1.0 / jnp.sqrt(jnp.float32(_D))
    xb = jnp.concatenate(
        [x, jnp.broadcast_to(tokens[None], (_B, _T, _D))], axis=1)
    ei = edge_index.astype(jnp.int32)

    mesh = plsc.VectorSubcoreMesh(core_axis_name="c", subcore_axis_name="s")
    adj = pl.kernel(
        _adj_sc,
        out_type=jax.ShapeDtypeStruct((_B, _GG), jnp.float32),
        mesh=mesh,
        scratch_types=[
            pltpu.VMEM((2, _E), jnp.int32),
            pltpu.VMEM((_GG,), jnp.float32),
        ],
        compiler_params=pltpu.CompilerParams(needs_layout_passes=False),
    )(ei)
    A3 = adj.reshape(_B, _GN, _GN)

    rcross = jnp.concatenate(
        [jnp.zeros((_D, _N), jnp.float32), tokens.T], axis=1)
    ccols = jnp.zeros((_GN, _GN), jnp.float32)
    ccols = ccols.at[:_N, _N:].set(1.0)

    wA1 = jnp.concatenate([Wq1.T * inv, Wk1.T, Wv1.T, Ws1.T], axis=1)
    bA1 = jnp.concatenate([bq1 * inv, bk1, bv1, bs1]).reshape(1, -1)
    wA2 = jnp.concatenate([Wq2.T * inv, Wk2.T, Wv2.T, Ws2.T], axis=1)
    bA2 = jnp.concatenate([bq2 * inv, bk2, bv2, bs2]).reshape(1, -1)

    args = [xb, A3, tokens, rcross, ccols,
            wA1, bA1, wA2, bA2,
            Wa.T / jnp.float32(_GN), ba.reshape(1, -1)]

    full = lambda i: (0, 0)
    in_specs = [
        pl.BlockSpec((_GB, _GN, _D), lambda i: (i, 0, 0)),
        pl.BlockSpec((_GB, _GN, _GN), lambda i: (i, 0, 0)),
        pl.BlockSpec((_T, _D), full),
        pl.BlockSpec((_D, _GN), full),
        pl.BlockSpec((_GN, _GN), full),
    ]
    for _ in range(2):
        in_specs.append(pl.BlockSpec((_D, 4 * _D), full))
        in_specs.append(pl.BlockSpec((1, 4 * _D), full))
    in_specs.append(pl.BlockSpec((_D, C), full))
    in_specs.append(pl.BlockSpec((1, C), full))

    f32 = jnp.float32
    scratch = [
        pltpu.VMEM((_R, 4 * _D), f32),
        pltpu.VMEM((_R, 2 * _D), f32),
        pltpu.VMEM((_R, _D), f32),
        pltpu.VMEM((_GB, _GN, _GN), f32),
        pltpu.VMEM((_GB, _D), f32),
    ]
    return pl.pallas_call(
        _kern,
        grid=(_NPROG,),
        in_specs=in_specs,
        out_specs=pl.BlockSpec((_GB, C), lambda i: (i, 0)),
        out_shape=jax.ShapeDtypeStruct((_B, C), jnp.float32),
        scratch_shapes=scratch,
        compiler_params=pltpu.CompilerParams(
            dimension_semantics=("parallel",)),
    )(*args)


# speedup vs baseline: 1.4000x; 1.0887x over previous
import jax
import jax.numpy as jnp
from jax.experimental import pallas as pl
from jax.experimental.pallas import tpu as pltpu
from jax.experimental.pallas import tpu_sc as plsc

_INNER_PRUNE = 0.3
_CROSS_PRUNE = 0.1
_HI = jax.lax.Precision.HIGHEST

_B = 128
_N = 78
_T = 10
_GN = _T + _N
_D = 128
_E = 1248
_GB = 8
_NPROG = _B // _GB
_R = _GB * _GN

_NC = 2
_NS = 16
_GPW = _B // (_NC * _NS)
_GG = _GN * _GN


def _adj_sc(ei_hbm, a_hbm, ei_v, acc_v):
    wid = jax.lax.axis_index("s") * _NC + jax.lax.axis_index("c")
    base = wid * _GPW
    ones = jnp.ones((16,), jnp.float32)
    zeros = jnp.zeros((16,), jnp.float32)
    for gg in range(_GPW):
        g = base + gg
        pltpu.sync_copy(ei_hbm.at[g], ei_v)

        def zbody(j, c):
            acc_v[pl.ds(pl.multiple_of(j * 16, 16), 16)] = zeros
            return c
        jax.lax.fori_loop(0, _GG // 16, zbody, 0)

        def ebody(j, c):
            o = pl.multiple_of(j * 16, 16)
            src = ei_v[0, pl.ds(o, 16)]
            dst = ei_v[1, pl.ds(o, 16)]
            plsc.addupdate_scatter(acc_v, [dst * _GN + src], ones)
            return c
        jax.lax.fori_loop(0, _E // 16, ebody, 0)

        pltpu.sync_copy(acc_v, a_hbm.at[g])


def _kern(x_ref, a_ref, tok_ref, rcross_ref, ccols_ref,
          wA1, bA1, wA2, bA2,
          waT, ba, out_ref,
          pscr, var, hr, lwr, pr, xs):
    tok = tok_ref[:]

    for g in range(_GB):
        r0 = g * _GN
        xs[r0:r0 + _N, :] = x_ref[g]
        xs[r0 + _N:r0 + _GN, :] = tok

    g_tt = jax.lax.dot_general(tok, tok, (((1,), (1,)), ((), ())),
                               precision=_HI)
    wtt = jnp.where(jax.nn.sigmoid(g_tt) >= _INNER_PRUNE, 1.0, 0.0)
    wttpad = jnp.pad(wtt, ((_N, 0), (_N, 0)))

    xf = xs[:]
    zc = jax.lax.dot(xf, rcross_ref[:], precision=_HI)
    cw = jnp.where(jax.nn.sigmoid(zc) >= _CROSS_PRUNE, 1.0, 0.0)
    cw = cw.reshape(_GB, _GN, _GN) * ccols_ref[:][None]
    wall = cw + wttpad[None]

    for g in range(_GB):
        lwr[g] = jnp.log(a_ref[g] + wall[g])

    var[:, _D:] = jnp.ones((_R, _D), jnp.float32)
    pscr[:] = jax.lax.dot(xf, wA1[:], precision=_HI) + bA1[:]
    var[:, :_D] = pscr[:, 2 * _D:3 * _D]

    for g in range(_GB):
        r0 = g * _GN
        qg = pscr[r0:r0 + _GN, 0:_D]
        kg = pscr[r0:r0 + _GN, _D:2 * _D]
        L = jax.lax.dot_general(qg, kg, (((1,), (1,)), ((), ())),
                                precision=_HI)
        ex = jnp.exp(L + lwr[g])
        ad = jax.lax.dot(ex, var[r0:r0 + _GN], precision=_HI)
        o = (ad[:, :_D] / jnp.maximum(ad[:, _D:], 1e-16)
             + pscr[r0:r0 + _GN, 3 * _D:])
        hr[r0:r0 + _GN, :] = jnp.where(o >= 0.0, o, 0.01 * o)

    hf = hr[:]
    pscr[:] = jax.lax.dot(hf, wA2[:], precision=_HI) + bA2[:]
    var[:, :_D] = pscr[:, 2 * _D:3 * _D]

    for g in range(_GB):
        r0 = g * _GN
        qg = pscr[r0:r0 + _GN, 0:_D]
        kg = pscr[r0:r0 + _GN, _D:2 * _D]
        L = jax.lax.dot_general(qg, kg, (((1,), (1,)), ((), ())),
                                precision=_HI)
        ex = jnp.exp(L + lwr[g])
        ad = jax.lax.dot(ex, var[r0:r0 + _GN], precision=_HI)
        o = (ad[:, :_D] / jnp.maximum(ad[:, _D:], 1e-16)
             + pscr[r0:r0 + _GN, 3 * _D:])
        pr[g:g + 1, :] = jnp.sum(o, axis=0, keepdims=True)

    Z = jax.lax.dot(pr[:], waT[:], precision=_HI) + ba[:]
    mz = jnp.max(Z, axis=1, keepdims=True)
    ez = jnp.exp(Z - mz)
    out_ref[:] = ez / jnp.sum(ez, axis=1, keepdims=True)


def kernel(x, edge_index, tokens,
           Wq1, bq1, Wk1, bk1, Wv1, bv1, Ws1, bs1,
           Wq2, bq2, Wk2, bk2, Wv2, bv2, Ws2, bs2, Wa, ba):
    C = Wa.shape[0]
    inv = 1.0 / jnp.sqrt(jnp.float32(_D))
    ei = edge_index.astype(jnp.int32)

    mesh = plsc.VectorSubcoreMesh(core_axis_name="c", subcore_axis_name="s")
    adj = pl.kernel(
        _adj_sc,
        out_type=jax.ShapeDtypeStruct((_B, _GG), jnp.float32),
        mesh=mesh,
        scratch_types=[
            pltpu.VMEM((2, _E), jnp.int32),
            pltpu.VMEM((_GG,), jnp.float32),
        ],
        compiler_params=pltpu.CompilerParams(needs_layout_passes=False),
    )(ei)
    A3 = adj.reshape(_B, _GN, _GN)

    rcross = jnp.concatenate(
        [jnp.zeros((_D, _N), jnp.float32), tokens.T], axis=1)
    ccols = jnp.zeros((_GN, _GN), jnp.float32)
    ccols = ccols.at[:_N, _N:].set(1.0)

    wA1 = jnp.concatenate([Wq1.T * inv, Wk1.T, Wv1.T, Ws1.T], axis=1)
    bA1 = jnp.concatenate([bq1 * inv, bk1, bv1, bs1]).reshape(1, -1)
    wA2 = jnp.concatenate([Wq2.T * inv, Wk2.T, Wv2.T, Ws2.T], axis=1)
    bA2 = jnp.concatenate([bq2 * inv, bk2, bv2, bs2]).reshape(1, -1)

    args = [x, A3, tokens, rcross, ccols,
            wA1, bA1, wA2, bA2,
            Wa.T / jnp.float32(_GN), ba.reshape(1, -1)]

    full = lambda i: (0, 0)
    in_specs = [
        pl.BlockSpec((_GB, _N, _D), lambda i: (i, 0, 0)),
        pl.BlockSpec((_GB, _GN, _GN), lambda i: (i, 0, 0)),
        pl.BlockSpec((_T, _D), full),
        pl.BlockSpec((_D, _GN), full),
        pl.BlockSpec((_GN, _GN), full),
    ]
    for _ in range(2):
        in_specs.append(pl.BlockSpec((_D, 4 * _D), full))
        in_specs.append(pl.BlockSpec((1, 4 * _D), full))
    in_specs.append(pl.BlockSpec((_D, C), full))
    in_specs.append(pl.BlockSpec((1, C), full))

    f32 = jnp.float32
    scratch = [
        pltpu.VMEM((_R, 4 * _D), f32),
        pltpu.VMEM((_R, 2 * _D), f32),
        pltpu.VMEM((_R, _D), f32),
        pltpu.VMEM((_GB, _GN, _GN), f32),
        pltpu.VMEM((_GB, _D), f32),
        pltpu.VMEM((_R, _D), f32),
    ]
    return pl.pallas_call(
        _kern,
        grid=(_NPROG,),
        in_specs=in_specs,
        out_specs=pl.BlockSpec((_GB, C), lambda i: (i, 0)),
        out_shape=jax.ShapeDtypeStruct((_B, C), jnp.float32),
        scratch_shapes=scratch,
        compiler_params=pltpu.CompilerParams(
            dimension_semantics=("parallel",)),
    )(*args)


# speedup vs baseline: 1.4803x; 1.0574x over previous
import jax
import jax.numpy as jnp
from jax.experimental import pallas as pl
from jax.experimental.pallas import tpu as pltpu
from jax.experimental.pallas import tpu_sc as plsc

_INNER_PRUNE = 0.3
_CROSS_PRUNE = 0.1
_HI = jax.lax.Precision.HIGHEST

_B = 128
_N = 78
_T = 10
_GN = _T + _N
_D = 128
_E = 1248
_GB = 16
_NPROG = _B // _GB
_R = _GB * _GN

_NC = 2
_NS = 16
_GPW = _B // (_NC * _NS)
_GG = _GN * _GN


def _adj_sc(ei_hbm, a_hbm, ei_v, acc_v):
    wid = jax.lax.axis_index("s") * _NC + jax.lax.axis_index("c")
    base = wid * _GPW
    ones = jnp.ones((16,), jnp.float32)
    zeros = jnp.zeros((16,), jnp.float32)
    for gg in range(_GPW):
        g = base + gg
        pltpu.sync_copy(ei_hbm.at[g], ei_v)

        def zbody(j, c):
            acc_v[pl.ds(pl.multiple_of(j * 16, 16), 16)] = zeros
            return c
        jax.lax.fori_loop(0, _GG // 16, zbody, 0)

        def ebody(j, c):
            o = pl.multiple_of(j * 16, 16)
            src = ei_v[0, pl.ds(o, 16)]
            dst = ei_v[1, pl.ds(o, 16)]
            plsc.addupdate_scatter(acc_v, [dst * _GN + src], ones)
            return c
        jax.lax.fori_loop(0, _E // 16, ebody, 0)

        pltpu.sync_copy(acc_v, a_hbm.at[g])


def _kern(x_ref, a_ref, tok_ref, rcross_ref, ccols_ref,
          wA1, bA1, wA2, bA2,
          waT, ba, out_ref,
          pscr, var, hr, lwr, pr, xs):
    tok = tok_ref[:]

    for g in range(_GB):
        r0 = g * _GN
        xs[r0:r0 + _N, :] = x_ref[g]
        xs[r0 + _N:r0 + _GN, :] = tok

    g_tt = jax.lax.dot_general(tok, tok, (((1,), (1,)), ((), ())),
                               precision=_HI)
    wtt = jnp.where(jax.nn.sigmoid(g_tt) >= _INNER_PRUNE, 1.0, 0.0)
    wttpad = jnp.pad(wtt, ((_N, 0), (_N, 0)))

    xf = xs[:]
    zc = jax.lax.dot(xf, rcross_ref[:], precision=_HI)
    cw = jnp.where(jax.nn.sigmoid(zc) >= _CROSS_PRUNE, 1.0, 0.0)
    cw = cw.reshape(_GB, _GN, _GN) * ccols_ref[:][None]
    wall = cw + wttpad[None]

    for g in range(_GB):
        lwr[g] = jnp.log(a_ref[g] + wall[g])

    var[:, _D:] = jnp.ones((_R, _D), jnp.float32)
    pscr[:] = jax.lax.dot(xf, wA1[:], precision=_HI) + bA1[:]
    var[:, :_D] = pscr[:, 2 * _D:3 * _D]

    for g in range(_GB):
        r0 = g * _GN
        qg = pscr[r0:r0 + _GN, 0:_D]
        kg = pscr[r0:r0 + _GN, _D:2 * _D]
        L = jax.lax.dot_general(qg, kg, (((1,), (1,)), ((), ())),
                                precision=_HI)
        ex = jnp.exp(L + lwr[g])
        ad = jax.lax.dot(ex, var[r0:r0 + _GN], precision=_HI)
        o = (ad[:, :_D] / jnp.maximum(ad[:, _D:], 1e-16)
             + pscr[r0:r0 + _GN, 3 * _D:])
        hr[r0:r0 + _GN, :] = jnp.where(o >= 0.0, o, 0.01 * o)

    hf = hr[:]
    pscr[:] = jax.lax.dot(hf, wA2[:], precision=_HI) + bA2[:]
    var[:, :_D] = pscr[:, 2 * _D:3 * _D]

    for g in range(_GB):
        r0 = g * _GN
        qg = pscr[r0:r0 + _GN, 0:_D]
        kg = pscr[r0:r0 + _GN, _D:2 * _D]
        L = jax.lax.dot_general(qg, kg, (((1,), (1,)), ((), ())),
                                precision=_HI)
        ex = jnp.exp(L + lwr[g])
        ad = jax.lax.dot(ex, var[r0:r0 + _GN], precision=_HI)
        o = (ad[:, :_D] / jnp.maximum(ad[:, _D:], 1e-16)
             + pscr[r0:r0 + _GN, 3 * _D:])
        pr[g:g + 1, :] = jnp.sum(o, axis=0, keepdims=True)

    Z = jax.lax.dot(pr[:], waT[:], precision=_HI) + ba[:]
    mz = jnp.max(Z, axis=1, keepdims=True)
    ez = jnp.exp(Z - mz)
    out_ref[:] = ez / jnp.sum(ez, axis=1, keepdims=True)


def kernel(x, edge_index, tokens,
           Wq1, bq1, Wk1, bk1, Wv1, bv1, Ws1, bs1,
           Wq2, bq2, Wk2, bk2, Wv2, bv2, Ws2, bs2, Wa, ba):
    C = Wa.shape[0]
    inv = 1.0 / jnp.sqrt(jnp.float32(_D))
    ei = edge_index.astype(jnp.int32)

    mesh = plsc.VectorSubcoreMesh(core_axis_name="c", subcore_axis_name="s")
    adj = pl.kernel(
        _adj_sc,
        out_type=jax.ShapeDtypeStruct((_B, _GG), jnp.float32),
        mesh=mesh,
        scratch_types=[
            pltpu.VMEM((2, _E), jnp.int32),
            pltpu.VMEM((_GG,), jnp.float32),
        ],
        compiler_params=pltpu.CompilerParams(needs_layout_passes=False),
    )(ei)
    A3 = adj.reshape(_B, _GN, _GN)

    rcross = jnp.concatenate(
        [jnp.zeros((_D, _N), jnp.float32), tokens.T], axis=1)
    ccols = jnp.zeros((_GN, _GN), jnp.float32)
    ccols = ccols.at[:_N, _N:].set(1.0)

    wA1 = jnp.concatenate([Wq1.T * inv, Wk1.T, Wv1.T, Ws1.T], axis=1)
    bA1 = jnp.concatenate([bq1 * inv, bk1, bv1, bs1]).reshape(1, -1)
    wA2 = jnp.concatenate([Wq2.T * inv, Wk2.T, Wv2.T, Ws2.T], axis=1)
    bA2 = jnp.concatenate([bq2 * inv, bk2, bv2, bs2]).reshape(1, -1)

    args = [x, A3, tokens, rcross, ccols,
            wA1, bA1, wA2, bA2,
            Wa.T / jnp.float32(_GN), ba.reshape(1, -1)]

    full = lambda i: (0, 0)
    in_specs = [
        pl.BlockSpec((_GB, _N, _D), lambda i: (i, 0, 0)),
        pl.BlockSpec((_GB, _GN, _GN), lambda i: (i, 0, 0)),
        pl.BlockSpec((_T, _D), full),
        pl.BlockSpec((_D, _GN), full),
        pl.BlockSpec((_GN, _GN), full),
    ]
    for _ in range(2):
        in_specs.append(pl.BlockSpec((_D, 4 * _D), full))
        in_specs.append(pl.BlockSpec((1, 4 * _D), full))
    in_specs.append(pl.BlockSpec((_D, C), full))
    in_specs.append(pl.BlockSpec((1, C), full))

    f32 = jnp.float32
    scratch = [
        pltpu.VMEM((_R, 4 * _D), f32),
        pltpu.VMEM((_R, 2 * _D), f32),
        pltpu.VMEM((_R, _D), f32),
        pltpu.VMEM((_GB, _GN, _GN), f32),
        pltpu.VMEM((_GB, _D), f32),
        pltpu.VMEM((_R, _D), f32),
    ]
    return pl.pallas_call(
        _kern,
        grid=(_NPROG,),
        in_specs=in_specs,
        out_specs=pl.BlockSpec((_GB, C), lambda i: (i, 0)),
        out_shape=jax.ShapeDtypeStruct((_B, C), jnp.float32),
        scratch_shapes=scratch,
        compiler_params=pltpu.CompilerParams(
            dimension_semantics=("parallel",)),
    )(*args)


# speedup vs baseline: 1.4907x; 1.0070x over previous
import jax
import jax.numpy as jnp
from jax.experimental import pallas as pl
from jax.experimental.pallas import tpu as pltpu
from jax.experimental.pallas import tpu_sc as plsc

_INNER_PRUNE = 0.3
_CROSS_PRUNE = 0.1
_HI = jax.lax.Precision.HIGHEST

_B = 128
_N = 78
_T = 10
_GN = _T + _N
_D = 128
_E = 1248
_GB = 32
_NPROG = _B // _GB
_R = _GB * _GN

_NC = 2
_NS = 16
_GPW = _B // (_NC * _NS)
_GG = _GN * _GN


def _adj_sc(ei_hbm, a_hbm, ei_v, acc_v):
    wid = jax.lax.axis_index("s") * _NC + jax.lax.axis_index("c")
    base = wid * _GPW
    ones = jnp.ones((16,), jnp.float32)
    zeros = jnp.zeros((16,), jnp.float32)
    for gg in range(_GPW):
        g = base + gg
        pltpu.sync_copy(ei_hbm.at[g], ei_v)

        def zbody(j, c):
            acc_v[pl.ds(pl.multiple_of(j * 16, 16), 16)] = zeros
            return c
        jax.lax.fori_loop(0, _GG // 16, zbody, 0)

        def ebody(j, c):
            o = pl.multiple_of(j * 16, 16)
            src = ei_v[0, pl.ds(o, 16)]
            dst = ei_v[1, pl.ds(o, 16)]
            plsc.addupdate_scatter(acc_v, [dst * _GN + src], ones)
            return c
        jax.lax.fori_loop(0, _E // 16, ebody, 0)

        pltpu.sync_copy(acc_v, a_hbm.at[g])


def _kern(x_ref, a_ref, tok_ref, rcross_ref, ccols_ref,
          wA1, bA1, wA2, bA2,
          waT, ba, out_ref,
          pscr, var, hr, lwr, pr, xs):
    tok = tok_ref[:]

    for g in range(_GB):
        r0 = g * _GN
        xs[r0:r0 + _N, :] = x_ref[g]
        xs[r0 + _N:r0 + _GN, :] = tok

    g_tt = jax.lax.dot_general(tok, tok, (((1,), (1,)), ((), ())),
                               precision=_HI)
    wtt = jnp.where(jax.nn.sigmoid(g_tt) >= _INNER_PRUNE, 1.0, 0.0)
    wttpad = jnp.pad(wtt, ((_N, 0), (_N, 0)))

    xf = xs[:]
    zc = jax.lax.dot(xf, rcross_ref[:], precision=_HI)
    cw = jnp.where(jax.nn.sigmoid(zc) >= _CROSS_PRUNE, 1.0, 0.0)
    cw = cw.reshape(_GB, _GN, _GN) * ccols_ref[:][None]
    wall = cw + wttpad[None]

    for g in range(_GB):
        lwr[g] = jnp.log(a_ref[g] + wall[g])

    var[:, _D:] = jnp.ones((_R, _D), jnp.float32)
    pscr[:] = jax.lax.dot(xf, wA1[:], precision=_HI) + bA1[:]
    var[:, :_D] = pscr[:, 2 * _D:3 * _D]

    for g in range(_GB):
        r0 = g * _GN
        qg = pscr[r0:r0 + _GN, 0:_D]
        kg = pscr[r0:r0 + _GN, _D:2 * _D]
        L = jax.lax.dot_general(qg, kg, (((1,), (1,)), ((), ())),
                                precision=_HI)
        ex = jnp.exp(L + lwr[g])
        ad = jax.lax.dot(ex, var[r0:r0 + _GN], precision=_HI)
        o = (ad[:, :_D] / jnp.maximum(ad[:, _D:], 1e-16)
             + pscr[r0:r0 + _GN, 3 * _D:])
        hr[r0:r0 + _GN, :] = jnp.where(o >= 0.0, o, 0.01 * o)

    hf = hr[:]
    pscr[:] = jax.lax.dot(hf, wA2[:], precision=_HI) + bA2[:]
    var[:, :_D] = pscr[:, 2 * _D:3 * _D]

    for g in range(_GB):
        r0 = g * _GN
        qg = pscr[r0:r0 + _GN, 0:_D]
        kg = pscr[r0:r0 + _GN, _D:2 * _D]
        L = jax.lax.dot_general(qg, kg, (((1,), (1,)), ((), ())),
                                precision=_HI)
        ex = jnp.exp(L + lwr[g])
        ad = jax.lax.dot(ex, var[r0:r0 + _GN], precision=_HI)
        o = (ad[:, :_D] / jnp.maximum(ad[:, _D:], 1e-16)
             + pscr[r0:r0 + _GN, 3 * _D:])
        pr[g:g + 1, :] = jnp.sum(o, axis=0, keepdims=True)

    Z = jax.lax.dot(pr[:], waT[:], precision=_HI) + ba[:]
    mz = jnp.max(Z, axis=1, keepdims=True)
    ez = jnp.exp(Z - mz)
    out_ref[:] = ez / jnp.sum(ez, axis=1, keepdims=True)


def kernel(x, edge_index, tokens,
           Wq1, bq1, Wk1, bk1, Wv1, bv1, Ws1, bs1,
           Wq2, bq2, Wk2, bk2, Wv2, bv2, Ws2, bs2, Wa, ba):
    C = Wa.shape[0]
    inv = 1.0 / jnp.sqrt(jnp.float32(_D))
    ei = edge_index.astype(jnp.int32)

    mesh = plsc.VectorSubcoreMesh(core_axis_name="c", subcore_axis_name="s")
    adj = pl.kernel(
        _adj_sc,
        out_type=jax.ShapeDtypeStruct((_B, _GG), jnp.float32),
        mesh=mesh,
        scratch_types=[
            pltpu.VMEM((2, _E), jnp.int32),
            pltpu.VMEM((_GG,), jnp.float32),
        ],
        compiler_params=pltpu.CompilerParams(needs_layout_passes=False),
    )(ei)
    A3 = adj.reshape(_B, _GN, _GN)

    rcross = jnp.concatenate(
        [jnp.zeros((_D, _N), jnp.float32), tokens.T], axis=1)
    ccols = jnp.zeros((_GN, _GN), jnp.float32)
    ccols = ccols.at[:_N, _N:].set(1.0)

    wA1 = jnp.concatenate([Wq1.T * inv, Wk1.T, Wv1.T, Ws1.T], axis=1)
    bA1 = jnp.concatenate([bq1 * inv, bk1, bv1, bs1]).reshape(1, -1)
    wA2 = jnp.concatenate([Wq2.T * inv, Wk2.T, Wv2.T, Ws2.T], axis=1)
    bA2 = jnp.concatenate([bq2 * inv, bk2, bv2, bs2]).reshape(1, -1)

    args = [x, A3, tokens, rcross, ccols,
            wA1, bA1, wA2, bA2,
            Wa.T / jnp.float32(_GN), ba.reshape(1, -1)]

    full = lambda i: (0, 0)
    in_specs = [
        pl.BlockSpec((_GB, _N, _D), lambda i: (i, 0, 0)),
        pl.BlockSpec((_GB, _GN, _GN), lambda i: (i, 0, 0)),
        pl.BlockSpec((_T, _D), full),
        pl.BlockSpec((_D, _GN), full),
        pl.BlockSpec((_GN, _GN), full),
    ]
    for _ in range(2):
        in_specs.append(pl.BlockSpec((_D, 4 * _D), full))
        in_specs.append(pl.BlockSpec((1, 4 * _D), full))
    in_specs.append(pl.BlockSpec((_D, C), full))
    in_specs.append(pl.BlockSpec((1, C), full))

    f32 = jnp.float32
    scratch = [
        pltpu.VMEM((_R, 4 * _D), f32),
        pltpu.VMEM((_R, 2 * _D), f32),
        pltpu.VMEM((_R, _D), f32),
        pltpu.VMEM((_GB, _GN, _GN), f32),
        pltpu.VMEM((_GB, _D), f32),
        pltpu.VMEM((_R, _D), f32),
    ]
    return pl.pallas_call(
        _kern,
        grid=(_NPROG,),
        in_specs=in_specs,
        out_specs=pl.BlockSpec((_GB, C), lambda i: (i, 0)),
        out_shape=jax.ShapeDtypeStruct((_B, C), jnp.float32),
        scratch_shapes=scratch,
        compiler_params=pltpu.CompilerParams(
            dimension_semantics=("parallel",)),
    )(*args)
